# Initial kernel scaffold; baseline (speedup 1.0000x reference)
import jax
import jax.numpy as jnp
from jax.experimental import pallas as pl


def kernel(positions, pe):
    raise NotImplementedError("write your pallas kernel here")


# speedup vs baseline: 1.6289x; 1.6289x over previous
import functools

import jax
import jax.numpy as jnp
from jax import lax
from jax.experimental import pallas as pl
from jax.experimental.pallas import tpu as pltpu
from jax.experimental.pallas import tpu_sc as plsc

EMBED = 384
B_TOTAL = 16 * 16384
NC, NS, L = 2, 16, 16
NW = NC * NS
B_PER_W = B_TOTAL // NW
CHUNK = 128
N_CHUNKS = B_PER_W // CHUNK
POS_CHUNK = 4096
N_POS_CHUNKS = B_PER_W // POS_CHUNK


@jax.jit
def _sc_gather(pos2d, pe):
    mesh = plsc.VectorSubcoreMesh(core_axis_name="c", subcore_axis_name="s")

    @functools.partial(
        pl.kernel,
        mesh=mesh,
        compiler_params=pltpu.CompilerParams(use_tc_tiling_on_sc=False),
        out_type=jax.ShapeDtypeStruct((B_TOTAL, EMBED), jnp.float32),
        scratch_types=[
            pltpu.VMEM((3, POS_CHUNK), jnp.int32),
            pltpu.VMEM((N_CHUNKS, CHUNK), jnp.int32),
            pltpu.VMEM((CHUNK, EMBED), jnp.float32),
            pltpu.SemaphoreType.DMA,
        ],
    )
    def k(pos_hbm, pe_hbm, out_hbm, pos_v, idx_v, rows_v, sem):
        wid = lax.axis_index("s") * NC + lax.axis_index("c")
        base = wid * B_PER_W
        vecs_per_chunk = CHUNK // L

        def pos_phase(t, carry):
            row0 = base + t * POS_CHUNK
            for comp in range(3):
                pltpu.sync_copy(
                    pos_hbm.at[comp, pl.ds(row0, POS_CHUNK)],
                    pos_v.at[comp])

            def idx_body(jj, c2):
                j = t * (POS_CHUNK // L) + jj
                sl = pl.ds(jj * L, L)
                d = pos_v[0, sl]
                h = pos_v[1, sl]
                w = pos_v[2, sl]
                lin = d * 1024 + h * 32 + w
                idx_v[j // vecs_per_chunk,
                      pl.ds((j % vecs_per_chunk) * L, L)] = lin
                return c2

            return lax.fori_loop(0, POS_CHUNK // L, idx_body, carry)

        lax.fori_loop(0, N_POS_CHUNKS, pos_phase, 0)

        def gather_body(c, carry):
            pltpu.async_copy(pe_hbm.at[idx_v.at[c]], rows_v, sem).wait()
            pltpu.sync_copy(rows_v, out_hbm.at[pl.ds(base + c * CHUNK, CHUNK)])
            return carry

        lax.fori_loop(0, N_CHUNKS, gather_body, 0)

    return k(pos2d, pe)


def kernel(positions, pe):
    pos_t = positions.reshape(-1, 3).astype(jnp.int32).T
    out = _sc_gather(pos_t, pe)
    return out.reshape(positions.shape[0], positions.shape[1], EMBED)


# speedup vs baseline: 1.7097x; 1.0496x over previous
import functools

import jax
import jax.numpy as jnp
from jax import lax
from jax.experimental import pallas as pl
from jax.experimental.pallas import tpu as pltpu
from jax.experimental.pallas import tpu_sc as plsc

EMBED = 384
B_TOTAL = 16 * 16384
NC, NS, L = 2, 16, 16
NW = NC * NS
B_PER_W = B_TOTAL // NW
CHUNK = 128
N_CHUNKS = B_PER_W // CHUNK
POS_CHUNK = 4096
N_POS_CHUNKS = B_PER_W // POS_CHUNK


@jax.jit
def _sc_gather(pos2d, pe):
    mesh = plsc.VectorSubcoreMesh(core_axis_name="c", subcore_axis_name="s")

    @functools.partial(
        pl.kernel,
        mesh=mesh,
        compiler_params=pltpu.CompilerParams(use_tc_tiling_on_sc=False),
        out_type=jax.ShapeDtypeStruct((B_TOTAL, EMBED), jnp.float32),
        scratch_types=[
            pltpu.VMEM((3, POS_CHUNK), jnp.int32),
            pltpu.VMEM((N_CHUNKS, CHUNK), jnp.int32),
            pltpu.VMEM((2, CHUNK, EMBED), jnp.float32),
            pltpu.SemaphoreType.DMA,
            pltpu.SemaphoreType.DMA,
        ],
    )
    def k(pos_hbm, pe_hbm, out_hbm, pos_v, idx_v, rows_v, gsem, ssem):
        wid = lax.axis_index("s") * NC + lax.axis_index("c")
        base = wid * B_PER_W
        vecs_per_chunk = CHUNK // L

        def pos_phase(t, carry):
            row0 = base + t * POS_CHUNK
            for comp in range(3):
                pltpu.sync_copy(
                    pos_hbm.at[comp, pl.ds(row0, POS_CHUNK)],
                    pos_v.at[comp])

            def idx_body(jj, c2):
                j = t * (POS_CHUNK // L) + jj
                sl = pl.ds(jj * L, L)
                d = pos_v[0, sl]
                h = pos_v[1, sl]
                w = pos_v[2, sl]
                lin = d * 1024 + h * 32 + w
                idx_v[j // vecs_per_chunk,
                      pl.ds((j % vecs_per_chunk) * L, L)] = lin
                return c2

            return lax.fori_loop(0, POS_CHUNK // L, idx_body, carry)

        lax.fori_loop(0, N_POS_CHUNKS, pos_phase, 0)

        def gather(c, p):
            pltpu.async_copy(pe_hbm.at[idx_v.at[c]], rows_v.at[p], gsem).wait()

        def start_store(c, p):
            pltpu.async_copy(
                rows_v.at[p], out_hbm.at[pl.ds(base + c * CHUNK, CHUNK)], ssem)

        def wait_one_store():
            pltpu.make_async_copy(
                rows_v.at[0], out_hbm.at[pl.ds(base, CHUNK)], ssem).wait()

        for c in range(2):
            gather(c, c)
            start_store(c, c)

        def gather_body(c, carry):
            p = lax.rem(c, 2)
            wait_one_store()
            gather(c, p)
            start_store(c, p)
            return carry

        lax.fori_loop(2, N_CHUNKS, gather_body, 0)
        wait_one_store()
        wait_one_store()

    return k(pos2d, pe)


def kernel(positions, pe):
    pos_t = positions.reshape(-1, 3).astype(jnp.int32).T
    out = _sc_gather(pos_t, pe)
    return out.reshape(positions.shape[0], positions.shape[1], EMBED)
